# baseline (speedup 1.0000x reference)
import functools

import jax
import jax.numpy as jnp
import numpy as np
from jax.experimental import pallas as pl
from jax.experimental.pallas import tpu as pltpu


def _attn_body(h, m0, wq, bq, wkv, bkv, wd2, bd, lnw, lnb, s2, s2t,
               out, *, num_l, inv_sqrt_dh):
    h0 = h[:, 0, :]
    q0 = (jnp.dot(h0, wq[...]) + bq[...]) * inv_sqrt_dh
    q2 = jnp.concatenate([q0, q0], axis=1)
    am0 = (m0[...] > 0).astype(jnp.float32)
    s2m = s2[...]
    s2tm = s2t[...]
    m = None
    d = None
    acc = None
    for l in range(num_l):
        kv = jnp.dot(h[:, l, :], wkv[...]) + bkv[...]
        att = jnp.dot(q2 * kv, s2m)
        if l == 0:
            att = att + (-1e4) * (1.0 - am0)
            m = att
            d = jnp.ones_like(att)
            acc = jnp.dot(d, s2tm) * kv
        else:
            att = att - 1e4
            m_new = jnp.maximum(m, att)
            alpha = jnp.exp(m - m_new)
            e = jnp.exp(att - m_new)
            m = m_new
            d = d * alpha + e
            acc = acc * jnp.dot(alpha, s2tm) + jnp.dot(e, s2tm) * kv

    denom = jnp.dot(d, s2tm)
    denom = denom + (denom == 0.0)
    ctx = acc / denom
    hs = jnp.dot(ctx, wd2[...]) + bd[...]
    x = hs + h0
    mu = jnp.mean(x, axis=1, keepdims=True)
    xc = x - mu
    var = jnp.mean(xc * xc, axis=1, keepdims=True)
    xn = xc / jnp.sqrt(var + 1e-12)
    out[...] = lnw[...] * xn + lnb[...]


def _scores_body(sel, emb, out):
    out[...] = jax.lax.dot_general(
        sel[...].astype(jnp.bfloat16), emb[...].astype(jnp.bfloat16),
        (((1,), (1,)), ((), ())),
        preferred_element_type=jnp.float32).astype(jnp.bfloat16)


def kernel(hidden, mask, time_delta, Wq, bq, Wk, bk, Wv, bv, Wd, bd, ln_w, ln_b, emb):
    B, L, D = hidden.shape
    V = emb.shape[0]
    H = 4
    DH = D // H

    m0 = mask[:, 0].reshape(B, 1)
    wkv = jnp.concatenate([Wk, Wv], axis=1)
    bkv = jnp.concatenate([bk, bv]).reshape(1, 2 * D)
    wd2 = jnp.concatenate([jnp.zeros_like(Wd), Wd], axis=0)
    eye = jnp.repeat(jnp.eye(H, dtype=jnp.float32), DH, axis=0)
    s2 = jnp.concatenate([eye, jnp.zeros_like(eye)], axis=0)
    s2t = jnp.concatenate([jnp.zeros_like(eye), eye], axis=0).T
    b2 = lambda v: v.reshape(1, D)

    NB = 2
    BB = B // NB
    const = lambda i: (0, 0)
    select = pl.pallas_call(
        functools.partial(_attn_body, num_l=L, inv_sqrt_dh=1.0 / np.sqrt(DH)),
        grid=(NB,),
        in_specs=[
            pl.BlockSpec((BB, L, D), lambda i: (i, 0, 0)),
            pl.BlockSpec((BB, 1), lambda i: (i, 0)),
            pl.BlockSpec((D, D), const),
            pl.BlockSpec((1, D), const),
            pl.BlockSpec((D, 2 * D), const),
            pl.BlockSpec((1, 2 * D), const),
            pl.BlockSpec((2 * D, D), const),
            pl.BlockSpec((1, D), const),
            pl.BlockSpec((1, D), const),
            pl.BlockSpec((1, D), const),
            pl.BlockSpec((2 * D, H), const),
            pl.BlockSpec((H, 2 * D), const),
        ],
        out_specs=pl.BlockSpec((BB, D), lambda i: (i, 0)),
        out_shape=jax.ShapeDtypeStruct((B, D), jnp.float32),
        compiler_params=pltpu.CompilerParams(
            dimension_semantics=("arbitrary",)),
    )(hidden, m0, Wq, b2(bq), wkv, bkv, wd2, b2(bd), b2(ln_w), b2(ln_b),
      s2, s2t)

    VB = 4096
    nvb = pl.cdiv(V, VB)
    scores = pl.pallas_call(
        _scores_body,
        grid=(nvb,),
        in_specs=[
            pl.BlockSpec((B, D), lambda j: (0, 0)),
            pl.BlockSpec((VB, D), lambda j: (j, 0)),
        ],
        out_specs=pl.BlockSpec((B, VB), lambda j: (0, j)),
        out_shape=jax.ShapeDtypeStruct((B, V), jnp.bfloat16),
        compiler_params=pltpu.CompilerParams(
            dimension_semantics=("arbitrary",)),
    )(select, emb)
    return scores.astype(jnp.float32)


# speedup vs baseline: 1.0459x; 1.0459x over previous
import functools

import jax
import jax.numpy as jnp
import numpy as np
from jax.experimental import pallas as pl
from jax.experimental.pallas import tpu as pltpu

_BF = jnp.bfloat16


def _attn_body(h, m0, wq, bq, wkv2, bkv2, wd2, bd, lnw, lnb, s4, et, out,
               *, num_l, inv_sqrt_dh):
    D = wq.shape[0]
    h0 = h[:, :D]
    q0 = (jnp.dot(h0.astype(_BF), wq[...].astype(_BF),
                  preferred_element_type=jnp.float32) + bq[...]) * inv_sqrt_dh
    q4 = jnp.concatenate([q0, q0, q0, q0], axis=1).astype(_BF)
    am0 = (m0[...] > 0).astype(jnp.float32)
    s4m = s4[...].astype(_BF)
    etm = et[...]
    eA, eB = etm[:4], etm[4:]
    both = eA + eB
    m = None
    d = None
    acc = None
    for p in range(num_l // 2):
        chunk = h[:, pl.dslice(p * 2 * D, 2 * D)].astype(_BF)
        kv2 = jnp.dot(chunk, wkv2[...],
                      preferred_element_type=jnp.float32) + bkv2[...]
        att2 = jnp.dot(q4 * kv2.astype(_BF), s4m,
                       preferred_element_type=jnp.float32)
        for half in range(2):
            l = 2 * p + half
            att = att2[:, half * 4:(half + 1) * 4]
            ee = (eA, eB)[half]
            if l == 0:
                att = att + (-1e4) * (1.0 - am0)
                m = att
                d = jnp.ones_like(att)
                acc = jnp.dot(d, ee) * kv2
            else:
                att = att - 1e4
                m_new = jnp.maximum(m, att)
                alpha = jnp.exp(m - m_new)
                e = jnp.exp(att - m_new)
                m = m_new
                d = d * alpha + e
                acc = acc * jnp.dot(alpha, both) + jnp.dot(e, ee) * kv2

    denom = jnp.dot(d, both)
    denom = denom + (denom == 0.0)
    ctx = acc / denom
    hs = jnp.dot(ctx.astype(_BF), wd2[...].astype(_BF),
                 preferred_element_type=jnp.float32) + bd[...]
    x = hs + h0
    mu = jnp.mean(x, axis=1, keepdims=True)
    xc = x - mu
    var = jnp.mean(xc * xc, axis=1, keepdims=True)
    xn = xc / jnp.sqrt(var + 1e-12)
    out[...] = lnw[...] * xn + lnb[...]


def _scores_body(sel, emb, out):
    out[...] = jax.lax.dot_general(
        sel[...].astype(_BF), emb[...].astype(_BF),
        (((1,), (1,)), ((), ())),
        preferred_element_type=jnp.float32).astype(_BF)


def kernel(hidden, mask, time_delta, Wq, bq, Wk, bk, Wv, bv, Wd, bd, ln_w, ln_b, emb):
    B, L, D = hidden.shape
    V = emb.shape[0]
    H = 4
    DH = D // H

    hflat = hidden.reshape(B, L * D)
    m0 = mask[:, 0].reshape(B, 1)
    wkv = jnp.concatenate([Wk, Wv], axis=1)
    z = jnp.zeros_like(wkv)
    wkv2 = jnp.concatenate(
        [jnp.concatenate([wkv, z], axis=1),
         jnp.concatenate([z, wkv], axis=1)], axis=0)
    bkv = jnp.concatenate([bk, bv])
    bkv2 = jnp.concatenate([bkv, bkv]).reshape(1, 4 * D)
    eye = jnp.repeat(jnp.eye(H, dtype=jnp.float32), DH, axis=0)
    zy = jnp.zeros_like(eye)
    s4 = jnp.concatenate(
        [jnp.concatenate([eye, zy], axis=1),
         jnp.concatenate([zy, zy], axis=1),
         jnp.concatenate([zy, eye], axis=1),
         jnp.concatenate([zy, zy], axis=1)], axis=0)
    etA = jnp.concatenate([zy, eye, zy, zy], axis=0).T
    etB = jnp.concatenate([zy, zy, zy, eye], axis=0).T
    et = jnp.concatenate([etA, etB], axis=0)
    wd4 = jnp.concatenate(
        [jnp.zeros_like(Wd), Wd, jnp.zeros_like(Wd), Wd], axis=0)
    b2 = lambda v: v.reshape(1, D)

    const = lambda i: (0, 0)
    select = pl.pallas_call(
        functools.partial(_attn_body, num_l=L, inv_sqrt_dh=1.0 / np.sqrt(DH)),
        grid=(1,),
        in_specs=[
            pl.BlockSpec((B, L * D), const),
            pl.BlockSpec((B, 1), const),
            pl.BlockSpec((D, D), const),
            pl.BlockSpec((1, D), const),
            pl.BlockSpec((2 * D, 4 * D), const),
            pl.BlockSpec((1, 4 * D), const),
            pl.BlockSpec((4 * D, D), const),
            pl.BlockSpec((1, D), const),
            pl.BlockSpec((1, D), const),
            pl.BlockSpec((1, D), const),
            pl.BlockSpec((4 * D, 2 * H), const),
            pl.BlockSpec((2 * H, 4 * D), const),
        ],
        out_specs=pl.BlockSpec((B, D), const),
        out_shape=jax.ShapeDtypeStruct((B, D), jnp.float32),
    )(hflat, m0, Wq, b2(bq), wkv2, bkv2, wd4, b2(bd), b2(ln_w), b2(ln_b),
      s4, et)

    VB = 4096
    nvb = pl.cdiv(V, VB)
    scores = pl.pallas_call(
        _scores_body,
        grid=(nvb,),
        in_specs=[
            pl.BlockSpec((B, D), lambda j: (0, 0)),
            pl.BlockSpec((VB, D), lambda j: (j, 0)),
        ],
        out_specs=pl.BlockSpec((B, VB), lambda j: (0, j)),
        out_shape=jax.ShapeDtypeStruct((B, V), _BF),
    )(select, emb)
    return scores.astype(jnp.float32)


# speedup vs baseline: 1.0466x; 1.0007x over previous
import functools

import jax
import jax.numpy as jnp
import numpy as np
from jax.experimental import pallas as pl
from jax.experimental.pallas import tpu as pltpu

_BF = jnp.bfloat16


def _attn_body(h, h0_ref, m0, wq, bq, wkv2, bkv2, wd2, bd, lnw, lnb, s4, et, out,
               *, num_l, inv_sqrt_dh):
    D = wq.shape[0]
    h0 = h0_ref[...]
    q0 = (jnp.dot(h0.astype(_BF), wq[...].astype(_BF),
                  preferred_element_type=jnp.float32) + bq[...]) * inv_sqrt_dh
    q4 = jnp.concatenate([q0, q0, q0, q0], axis=1).astype(_BF)
    am0 = (m0[...] > 0).astype(jnp.float32)
    s4m = s4[...].astype(_BF)
    etm = et[...]
    eA, eB = etm[:4], etm[4:]
    both = eA + eB
    m = None
    d = None
    acc = None
    for p in range(num_l // 2):
        chunk = h[:, pl.dslice(p * 2 * D, 2 * D)]
        kv2 = jnp.dot(chunk, wkv2[...],
                      preferred_element_type=jnp.float32) + bkv2[...]
        att2 = jnp.dot(q4 * kv2.astype(_BF), s4m,
                       preferred_element_type=jnp.float32)
        for half in range(2):
            l = 2 * p + half
            att = att2[:, half * 4:(half + 1) * 4]
            ee = (eA, eB)[half]
            if l == 0:
                att = att + (-1e4) * (1.0 - am0)
                m = att
                d = jnp.ones_like(att)
                acc = jnp.dot(d, ee) * kv2
            else:
                att = att - 1e4
                m_new = jnp.maximum(m, att)
                alpha = jnp.exp(m - m_new)
                e = jnp.exp(att - m_new)
                m = m_new
                d = d * alpha + e
                acc = acc * jnp.dot(alpha, both) + jnp.dot(e, ee) * kv2

    denom = jnp.dot(d, both)
    denom = denom + (denom == 0.0)
    ctx = acc / denom
    hs = jnp.dot(ctx.astype(_BF), wd2[...].astype(_BF),
                 preferred_element_type=jnp.float32) + bd[...]
    x = hs + h0
    mu = jnp.mean(x, axis=1, keepdims=True)
    xc = x - mu
    var = jnp.mean(xc * xc, axis=1, keepdims=True)
    xn = xc / jnp.sqrt(var + 1e-12)
    out[...] = (lnw[...] * xn + lnb[...]).astype(_BF)


def _scores_body(sel, emb, out):
    out[...] = jax.lax.dot_general(
        sel[...], emb[...], (((1,), (1,)), ((), ())),
        preferred_element_type=jnp.float32).astype(_BF)


def kernel(hidden, mask, time_delta, Wq, bq, Wk, bk, Wv, bv, Wd, bd, ln_w, ln_b, emb):
    B, L, D = hidden.shape
    V = emb.shape[0]
    H = 4
    DH = D // H

    hflat = hidden.astype(_BF).reshape(B, L * D)
    h0 = hidden[:, 0, :]
    m0 = mask[:, 0].reshape(B, 1)
    wkv = jnp.concatenate([Wk, Wv], axis=1)
    z = jnp.zeros_like(wkv)
    wkv2 = jnp.concatenate(
        [jnp.concatenate([wkv, z], axis=1),
         jnp.concatenate([z, wkv], axis=1)], axis=0)
    bkv = jnp.concatenate([bk, bv])
    bkv2 = jnp.concatenate([bkv, bkv]).reshape(1, 4 * D)
    eye = jnp.repeat(jnp.eye(H, dtype=jnp.float32), DH, axis=0)
    zy = jnp.zeros_like(eye)
    s4 = jnp.concatenate(
        [jnp.concatenate([eye, zy], axis=1),
         jnp.concatenate([zy, zy], axis=1),
         jnp.concatenate([zy, eye], axis=1),
         jnp.concatenate([zy, zy], axis=1)], axis=0)
    etA = jnp.concatenate([zy, eye, zy, zy], axis=0).T
    etB = jnp.concatenate([zy, zy, zy, eye], axis=0).T
    et = jnp.concatenate([etA, etB], axis=0)
    wd4 = jnp.concatenate(
        [jnp.zeros_like(Wd), Wd, jnp.zeros_like(Wd), Wd], axis=0)
    b2 = lambda v: v.reshape(1, D)

    const = lambda i: (0, 0)
    NB = 2
    BB = B // NB
    select = pl.pallas_call(
        functools.partial(_attn_body, num_l=L, inv_sqrt_dh=1.0 / np.sqrt(DH)),
        grid=(NB,),
        in_specs=[
            pl.BlockSpec((BB, L * D), lambda i: (i, 0)),
            pl.BlockSpec((BB, D), lambda i: (i, 0)),
            pl.BlockSpec((BB, 1), lambda i: (i, 0)),
            pl.BlockSpec((D, D), const),
            pl.BlockSpec((1, D), const),
            pl.BlockSpec((2 * D, 4 * D), const),
            pl.BlockSpec((1, 4 * D), const),
            pl.BlockSpec((4 * D, D), const),
            pl.BlockSpec((1, D), const),
            pl.BlockSpec((1, D), const),
            pl.BlockSpec((1, D), const),
            pl.BlockSpec((4 * D, 2 * H), const),
            pl.BlockSpec((2 * H, 4 * D), const),
        ],
        out_specs=pl.BlockSpec((BB, D), lambda i: (i, 0)),
        out_shape=jax.ShapeDtypeStruct((B, D), _BF),
    )(hflat, h0, m0, Wq, b2(bq), wkv2, bkv2, wd4, b2(bd), b2(ln_w), b2(ln_b),
      s4, et)

    VB = 8192
    nvb = pl.cdiv(V, VB)
    scores = pl.pallas_call(
        _scores_body,
        grid=(nvb,),
        in_specs=[
            pl.BlockSpec((B, D), lambda j: (0, 0)),
            pl.BlockSpec((VB, D), lambda j: (j, 0)),
        ],
        out_specs=pl.BlockSpec((B, VB), lambda j: (0, j)),
        out_shape=jax.ShapeDtypeStruct((B, V), _BF),
    )(select, emb.astype(_BF))
    return scores.astype(jnp.float32)
